# baseline (speedup 1.0000x reference)
import functools

import jax
import jax.numpy as jnp
from jax import lax
from jax.experimental import pallas as pl
from jax.experimental.pallas import tpu as pltpu
from jax.experimental.pallas import tpu_sc as plsc

N, E, DIN, DH, DOUT = 10000, 320000, 128, 128, 64

NC, NS = 2, 16
NW = NC * NS
C = 128
CH = 80
EPW = CH * C
E_PAD = EPW * NW
N_PAD = 10112
ROWS_PER_TILE = N_PAD // NS


def _sc_segment_sum(h, src_p, dst_p, zinit):
    mesh = plsc.VectorSubcoreMesh(core_axis_name="c", subcore_axis_name="s")

    @functools.partial(
        pl.kernel,
        out_type=jax.ShapeDtypeStruct((NC, N_PAD, DH), jnp.float32),
        mesh=mesh,
        scratch_types=[
            pltpu.VMEM((C,), jnp.int32),
            pltpu.VMEM((C,), jnp.int32),
            pltpu.VMEM((C,), jnp.int32),
            pltpu.VMEM((C,), jnp.int32),
            pltpu.VMEM((C, DH), jnp.float32),
            pltpu.VMEM((C, DH), jnp.float32),
            pltpu.VMEM_SHARED((N_PAD, DH), jnp.float32),
            pltpu.SemaphoreType.DMA,
            pltpu.SemaphoreType.DMA,
            pltpu.SemaphoreType.DMA,
            pltpu.SemaphoreType.DMA,
        ],
    )
    def k(h_hbm, src_hbm, dst_hbm, z_hbm, out_hbm,
          sA, dA, sB, dB, rA, rB, acc, isemA, isemB, gsemA, gsemB):
        cid = lax.axis_index("c")
        sid = lax.axis_index("s")
        wid = sid * NC + cid
        r0 = sid * ROWS_PER_TILE
        base = wid * EPW

        def issue_idx(i, sv, dv, sem):
            off = base + jnp.minimum(i, CH - 1) * C
            pltpu.async_copy(src_hbm.at[pl.ds(off, C)], sv, sem)
            pltpu.async_copy(dst_hbm.at[pl.ds(off, C)], dv, sem)

        def wait_idx(sv, dv, sem):
            pltpu.make_async_copy(src_hbm.at[pl.ds(0, C)], sv, sem).wait()
            pltpu.make_async_copy(dst_hbm.at[pl.ds(0, C)], dv, sem).wait()

        def wait_rows(rv, sem):
            pltpu.make_async_copy(h_hbm.at[pl.ds(0, C)], rv, sem).wait()

        issue_idx(0, sA, dA, isemA)
        issue_idx(1, sB, dB, isemB)
        pltpu.sync_copy(z_hbm.at[pl.ds(r0, ROWS_PER_TILE)],
                        acc.at[pl.ds(r0, ROWS_PER_TILE)])
        wait_idx(sA, dA, isemA)
        pltpu.async_copy(h_hbm.at[sA], rA, gsemA)
        plsc.subcore_barrier()

        def body(j, carry):
            i0 = 2 * j
            wait_idx(sB, dB, isemB)
            pltpu.async_copy(h_hbm.at[sB], rB, gsemB)
            wait_rows(rA, gsemA)
            pltpu.sync_copy(rA, acc.at[dA], add=True)
            issue_idx(i0 + 2, sA, dA, isemA)
            wait_rows(rB, gsemB)
            pltpu.sync_copy(rB, acc.at[dB], add=True)
            wait_idx(sA, dA, isemA)
            pltpu.async_copy(h_hbm.at[sA], rA, gsemA)
            issue_idx(i0 + 3, sB, dB, isemB)
            return carry

        lax.fori_loop(0, CH // 2, body, 0)
        wait_rows(rA, gsemA)
        wait_idx(sB, dB, isemB)
        plsc.subcore_barrier()
        pltpu.sync_copy(acc.at[pl.ds(r0, ROWS_PER_TILE)],
                        out_hbm.at[cid, pl.ds(r0, ROWS_PER_TILE)])

    return k(h, src_p, dst_p, zinit)


def _tc_layer(h, parts, W1, b1, W2, b2, g, be):
    def body(h_ref, p_ref, W1_ref, b1_ref, W2_ref, b2_ref, g_ref, be_ref, o_ref):
        agg = p_ref[0, :N, :] + p_ref[1, :N, :]
        h2 = h_ref[...] + agg
        a1 = jnp.maximum(
            jnp.dot(h2, W1_ref[...], preferred_element_type=jnp.float32)
            + b1_ref[...], 0.0)
        a2 = jnp.dot(a1, W2_ref[...], preferred_element_type=jnp.float32) \
            + b2_ref[...]
        a3 = jnp.maximum(a2, 0.0)
        mean = jnp.mean(a3, axis=0, keepdims=True)
        var = jnp.mean((a3 - mean) ** 2, axis=0, keepdims=True)
        o_ref[...] = g_ref[...] * (a3 - mean) * lax.rsqrt(var + 1e-5) \
            + be_ref[...]

    return pl.pallas_call(
        body,
        out_shape=jax.ShapeDtypeStruct((N, DH), jnp.float32),
    )(h, parts, W1, b1.reshape(1, DH), W2, b2.reshape(1, DH),
      g.reshape(1, DH), be.reshape(1, DH))


def _tc_head(h, fc1_W, fc1_b, fc2_W, fc2_b):
    def body(h_ref, W1_ref, b1_ref, W2_ref, b2_ref, o_ref):
        a1 = jnp.maximum(
            jnp.dot(h_ref[...], W1_ref[...], preferred_element_type=jnp.float32)
            + b1_ref[...], 0.0)
        z = jnp.dot(a1, W2_ref[...], preferred_element_type=jnp.float32) \
            + b2_ref[...]
        m = jnp.max(z, axis=-1, keepdims=True)
        ez = jnp.exp(z - m)
        lse = jnp.log(jnp.sum(ez, axis=-1, keepdims=True)) + m
        o_ref[...] = z - lse

    return pl.pallas_call(
        body,
        out_shape=jax.ShapeDtypeStruct((N, DOUT), jnp.float32),
    )(h, fc1_W, fc1_b.reshape(1, DH), fc2_W, fc2_b.reshape(1, DOUT))


def kernel(x, edge_index, l1_W1, l1_b1, l1_W2, l1_b2, l1_g, l1_be, l2_W1, l2_b1, l2_W2, l2_b2, l2_g, l2_be, l3_W1, l3_b1, l3_W2, l3_b2, l3_g, l3_be, fc1_W, fc1_b, fc2_W, fc2_b):
    pad = E_PAD - E
    src_p = jnp.concatenate([edge_index[0], jnp.zeros((pad,), jnp.int32)])
    dst_p = jnp.concatenate([edge_index[1], jnp.full((pad,), N, jnp.int32)])
    zinit = jnp.zeros((N_PAD, DH), jnp.float32)

    layers = [
        (l1_W1, l1_b1, l1_W2, l1_b2, l1_g, l1_be),
        (l2_W1, l2_b1, l2_W2, l2_b2, l2_g, l2_be),
        (l3_W1, l3_b1, l3_W2, l3_b2, l3_g, l3_be),
    ]
    h = x
    for (W1, b1, W2, b2, g, be) in layers:
        parts = _sc_segment_sum(h, src_p, dst_p, zinit)
        h = _tc_layer(h, parts, W1, b1, W2, b2, g, be)
    return _tc_head(h, fc1_W, fc1_b, fc2_W, fc2_b)


# speedup vs baseline: 1.0003x; 1.0003x over previous
import functools

import jax
import jax.numpy as jnp
from jax import lax
from jax.experimental import pallas as pl
from jax.experimental.pallas import tpu as pltpu
from jax.experimental.pallas import tpu_sc as plsc

N, E, DIN, DH, DOUT = 10000, 320000, 128, 128, 64

NC, NS = 2, 16
NW = NC * NS
C = 128
CH = 80
EPW = CH * C
E_PAD = EPW * NW
N_PAD = 10112
ROWS_PER_TILE = N_PAD // NS


def _sc_segment_sum(h, src_p, dst_p, zinit):
    mesh = plsc.VectorSubcoreMesh(core_axis_name="c", subcore_axis_name="s")

    @functools.partial(
        pl.kernel,
        out_type=jax.ShapeDtypeStruct((NC, N_PAD, DH), jnp.float32),
        mesh=mesh,
        scratch_types=[
            pltpu.VMEM((C,), jnp.int32),
            pltpu.VMEM((C,), jnp.int32),
            pltpu.VMEM((C,), jnp.int32),
            pltpu.VMEM((C,), jnp.int32),
            pltpu.VMEM((C, DH), jnp.float32),
            pltpu.VMEM((C, DH), jnp.float32),
            pltpu.VMEM_SHARED((N_PAD, DH), jnp.float32),
            pltpu.SemaphoreType.DMA,
            pltpu.SemaphoreType.DMA,
            pltpu.SemaphoreType.DMA,
            pltpu.SemaphoreType.DMA,
        ],
    )
    def k(h_hbm, src_hbm, dst_hbm, z_hbm, out_hbm,
          sA, dA, sB, dB, rA, rB, acc, isemA, isemB, gsemA, gsemB):
        cid = lax.axis_index("c")
        sid = lax.axis_index("s")
        wid = sid * NC + cid
        r0 = sid * ROWS_PER_TILE
        base = wid * EPW

        def issue_idx(i, sv, dv, sem):
            off = base + jnp.minimum(i, CH - 1) * C
            pltpu.async_copy(src_hbm.at[pl.ds(off, C)], sv, sem)
            pltpu.async_copy(dst_hbm.at[pl.ds(off, C)], dv, sem)

        def wait_idx(sv, dv, sem):
            pltpu.make_async_copy(src_hbm.at[pl.ds(0, C)], sv, sem).wait()
            pltpu.make_async_copy(dst_hbm.at[pl.ds(0, C)], dv, sem).wait()

        def wait_rows(rv, sem):
            pltpu.make_async_copy(h_hbm.at[pl.ds(0, C)], rv, sem).wait()

        issue_idx(0, sA, dA, isemA)
        issue_idx(1, sB, dB, isemB)
        pltpu.sync_copy(z_hbm.at[pl.ds(r0, ROWS_PER_TILE)],
                        acc.at[pl.ds(r0, ROWS_PER_TILE)])
        wait_idx(sA, dA, isemA)
        pltpu.async_copy(h_hbm.at[sA], rA, gsemA)
        plsc.subcore_barrier()

        def body(j, carry):
            i0 = 2 * j
            wait_idx(sB, dB, isemB)
            pltpu.async_copy(h_hbm.at[sB], rB, gsemB)
            wait_rows(rA, gsemA)
            pltpu.sync_copy(rA, acc.at[dA], add=True)
            issue_idx(i0 + 2, sA, dA, isemA)
            wait_rows(rB, gsemB)
            pltpu.sync_copy(rB, acc.at[dB], add=True)
            wait_idx(sA, dA, isemA)
            pltpu.async_copy(h_hbm.at[sA], rA, gsemA)
            issue_idx(i0 + 3, sB, dB, isemB)
            return carry

        lax.fori_loop(0, CH // 2, body, 0)
        wait_rows(rA, gsemA)
        wait_idx(sB, dB, isemB)
        plsc.subcore_barrier()
        pltpu.sync_copy(acc.at[pl.ds(r0, ROWS_PER_TILE)],
                        out_hbm.at[cid, pl.ds(r0, ROWS_PER_TILE)])

    return k(h, src_p, dst_p, zinit)


def _tc_layer(h, parts, W1, b1, W2, b2, g, be):
    def body(h_ref, p_ref, W1_ref, b1_ref, W2_ref, b2_ref, g_ref, be_ref, o_ref):
        agg = p_ref[0, :N, :] + p_ref[1, :N, :]
        h2 = h_ref[...] + agg
        a1 = jnp.maximum(
            jnp.dot(h2, W1_ref[...], preferred_element_type=jnp.float32)
            + b1_ref[...], 0.0)
        a2 = jnp.dot(a1, W2_ref[...], preferred_element_type=jnp.float32) \
            + b2_ref[...]
        a3 = jnp.maximum(a2, 0.0)
        mean = jnp.mean(a3, axis=0, keepdims=True)
        var = jnp.mean((a3 - mean) ** 2, axis=0, keepdims=True)
        o_ref[...] = g_ref[...] * (a3 - mean) * lax.rsqrt(var + 1e-5) \
            + be_ref[...]

    return pl.pallas_call(
        body,
        out_shape=jax.ShapeDtypeStruct((N, DH), jnp.float32),
    )(h, parts, W1, b1.reshape(1, DH), W2, b2.reshape(1, DH),
      g.reshape(1, DH), be.reshape(1, DH))


def _tc_head(h, fc1_W, fc1_b, fc2_W, fc2_b):
    def body(h_ref, W1_ref, b1_ref, W2_ref, b2_ref, o_ref):
        a1 = jnp.maximum(
            jnp.dot(h_ref[...], W1_ref[...], preferred_element_type=jnp.float32)
            + b1_ref[...], 0.0)
        z = jnp.dot(a1, W2_ref[...], preferred_element_type=jnp.float32) \
            + b2_ref[...]
        m = jnp.max(z, axis=-1, keepdims=True)
        ez = jnp.exp(z - m)
        lse = jnp.log(jnp.sum(ez, axis=-1, keepdims=True)) + m
        o_ref[...] = z - lse

    return pl.pallas_call(
        body,
        out_shape=jax.ShapeDtypeStruct((N, DOUT), jnp.float32),
    )(h, fc1_W, fc1_b.reshape(1, DH), fc2_W, fc2_b.reshape(1, DOUT))


def kernel(x, edge_index, l1_W1, l1_b1, l1_W2, l1_b2, l1_g, l1_be, l2_W1, l2_b1, l2_W2, l2_b2, l2_g, l2_be, l3_W1, l3_b1, l3_W2, l3_b2, l3_g, l3_be, fc1_W, fc1_b, fc2_W, fc2_b):
    pad = E_PAD - E
    src_p = jnp.concatenate([edge_index[0], jnp.zeros((pad,), jnp.int32)])
    junk = N + jnp.arange(pad, dtype=jnp.int32) % (N_PAD - N)
    dst_p = jnp.concatenate([edge_index[1], junk])
    zinit = jnp.zeros((N_PAD, DH), jnp.float32)

    layers = [
        (l1_W1, l1_b1, l1_W2, l1_b2, l1_g, l1_be),
        (l2_W1, l2_b1, l2_W2, l2_b2, l2_g, l2_be),
        (l3_W1, l3_b1, l3_W2, l3_b2, l3_g, l3_be),
    ]
    h = x
    for (W1, b1, W2, b2, g, be) in layers:
        parts = _sc_segment_sum(h, src_p, dst_p, zinit)
        h = _tc_layer(h, parts, W1, b1, W2, b2, g, be)
    return _tc_head(h, fc1_W, fc1_b, fc2_W, fc2_b)
